# baseline (speedup 1.0000x reference)
import functools

import jax
import jax.numpy as jnp
from jax import lax
from jax.experimental import pallas as pl
from jax.experimental.pallas import tpu as pltpu
from jax.experimental.pallas import tpu_sc as plsc

_HEADS = 12


def _make_reduce_rank_body(BH, SB, nsb):
    def body(a_ref, rank_ref, acc_ref):
        t = pl.program_id(0)

        @pl.when(t < BH)
        def _phase_a():
            m = t
            b = m // _HEADS
            data = a_ref[0]
            R, S = data.shape
            rows = jax.lax.broadcasted_iota(jnp.int32, (R, S), 0)
            cols = jax.lax.broadcasted_iota(jnp.int32, (R, S), 1)
            contrib = jnp.where(rows == cols, 0.0, data)
            part = jnp.sum(contrib, axis=0, keepdims=True)

            @pl.when(m % _HEADS == 0)
            def _():
                acc_ref[pl.ds(b, 1), :] = part

            @pl.when(m % _HEADS != 0)
            def _():
                acc_ref[pl.ds(b, 1), :] += part

        @pl.when(t >= BH)
        def _phase_b():
            i = t - BH
            b = i // nsb
            sblk = i % nsb
            S = acc_ref.shape[1]
            a_row = acc_ref[pl.ds(b, 1), :]
            a_col = jnp.transpose(
                acc_ref[pl.ds(b, 1), pl.ds(sblk * SB, SB)])
            s_glob = jax.lax.broadcasted_iota(jnp.int32, (SB, S), 0) + sblk * SB
            j_glob = jax.lax.broadcasted_iota(jnp.int32, (SB, S), 1)
            gt = a_row > a_col
            tie = jnp.logical_and(a_row == a_col, j_glob < s_glob)
            cmp = jnp.where(jnp.logical_or(gt, tie), 1.0, 0.0)
            rank_ref[...] = jnp.sum(cmp, axis=1, keepdims=True).astype(jnp.int32)

    return body


def _mul_body(gate_ref, x_ref, out_ref):
    out_ref[0] = x_ref[0] * gate_ref[...]


def _sc_gather(rank_flat, w):
    BS = rank_flat.shape[0]
    S = w.shape[0]
    info = plsc.get_sparse_core_info()
    NC, NS, L = info.num_cores, info.num_subcores, info.num_lanes
    NW = NC * NS
    per_w = BS // NW

    mesh = plsc.VectorSubcoreMesh(core_axis_name="c", subcore_axis_name="s")

    @functools.partial(
        pl.kernel,
        out_type=jax.ShapeDtypeStruct((BS,), jnp.float32),
        mesh=mesh,
        scratch_types=[
            pltpu.VMEM((S,), jnp.float32),
            pltpu.VMEM((per_w,), jnp.int32),
            pltpu.VMEM((per_w,), jnp.float32),
        ],
        compiler_params=pltpu.CompilerParams(needs_layout_passes=False),
    )
    def k(rank_hbm, w_hbm, gate_hbm, w_v, r_v, g_v):
        wid = lax.axis_index("s") * NC + lax.axis_index("c")
        base = wid * per_w
        pltpu.sync_copy(w_hbm, w_v)
        pltpu.sync_copy(rank_hbm.at[pl.ds(base, per_w)], r_v)
        for v in range(per_w // L):
            idx = r_v[pl.ds(v * L, L)]
            g_v[pl.ds(v * L, L)] = plsc.load_gather(w_v, [idx])
        pltpu.sync_copy(g_v, gate_hbm.at[pl.ds(base, per_w)])

    return k(rank_flat, w)


def kernel(x, atten, W):
    B, S, D = x.shape
    BH = atten.shape[0]
    SB = 1024
    nsb = S // SB
    nsteps = BH + B * nsb

    rank = pl.pallas_call(
        _make_reduce_rank_body(BH, SB, nsb),
        grid=(nsteps,),
        in_specs=[
            pl.BlockSpec(
                (1, S, S), lambda t, _m=BH - 1: (jnp.minimum(t, _m), 0, 0)),
        ],
        out_specs=pl.BlockSpec(
            (SB, 1), lambda t, _b=BH: (jnp.maximum(t - _b, 0), 0)),
        out_shape=jax.ShapeDtypeStruct((B * S, 1), jnp.int32),
        scratch_shapes=[pltpu.VMEM((B, S), jnp.float32)],
    )(atten)

    gate = _sc_gather(rank.reshape(B * S), W)
    gate_col = gate.reshape(B * S, 1)

    out = pl.pallas_call(
        _mul_body,
        grid=(B, nsb),
        in_specs=[
            pl.BlockSpec((SB, 1), lambda b, s, _n=nsb: (b * _n + s, 0)),
            pl.BlockSpec((1, SB, D), lambda b, s: (b, s, 0)),
        ],
        out_specs=pl.BlockSpec((1, SB, D), lambda b, s: (b, s, 0)),
        out_shape=jax.ShapeDtypeStruct((B, S, D), jnp.float32),
    )(gate_col, x)
    return out


# speedup vs baseline: 1.0155x; 1.0155x over previous
import functools

import jax
import jax.numpy as jnp
from jax import lax
from jax.experimental import pallas as pl
from jax.experimental.pallas import tpu as pltpu
from jax.experimental.pallas import tpu_sc as plsc

_HEADS = 12


def _make_reduce_rank_body(BH, SB, nsb):
    def body(a_ref, rank_ref, acc_ref):
        t = pl.program_id(0)

        @pl.when(t < BH)
        def _phase_a():
            m = t
            b = m // _HEADS
            data = a_ref[0]
            R, S = data.shape
            rows = jax.lax.broadcasted_iota(jnp.int32, (R, S), 0)
            cols = jax.lax.broadcasted_iota(jnp.int32, (R, S), 1)
            contrib = jnp.where(rows == cols, 0.0, data)
            part = jnp.sum(contrib, axis=0, keepdims=True)

            @pl.when(m % _HEADS == 0)
            def _():
                acc_ref[pl.ds(b, 1), :] = part

            @pl.when(m % _HEADS != 0)
            def _():
                acc_ref[pl.ds(b, 1), :] += part

        @pl.when(t >= BH)
        def _phase_b():
            i = t - BH
            b = i // nsb
            sblk = i % nsb
            S = acc_ref.shape[1]
            a_row = acc_ref[pl.ds(b, 1), :]
            a_col = jnp.transpose(
                acc_ref[pl.ds(b, 1), pl.ds(sblk * SB, SB)])
            s_glob = jax.lax.broadcasted_iota(jnp.int32, (SB, S), 0) + sblk * SB
            j_glob = jax.lax.broadcasted_iota(jnp.int32, (SB, S), 1)
            gt = a_row > a_col
            tie = jnp.logical_and(a_row == a_col, j_glob < s_glob)
            cmp = jnp.where(jnp.logical_or(gt, tie), 1.0, 0.0)
            rank_ref[...] = jnp.sum(cmp, axis=1, keepdims=True).astype(jnp.int32)

    return body


def _mul_body(gate_ref, x_ref, out_ref):
    out_ref[0] = x_ref[0] * gate_ref[...]


def _sc_gather(rank_flat, w):
    BS = rank_flat.shape[0]
    S = w.shape[0]
    info = plsc.get_sparse_core_info()
    NC, NS, L = info.num_cores, info.num_subcores, info.num_lanes
    NW = NC * NS
    per_w = BS // NW

    mesh = plsc.VectorSubcoreMesh(core_axis_name="c", subcore_axis_name="s")

    @functools.partial(
        pl.kernel,
        out_type=jax.ShapeDtypeStruct((BS,), jnp.float32),
        mesh=mesh,
        scratch_types=[
            pltpu.VMEM((S,), jnp.float32),
            pltpu.VMEM((per_w,), jnp.int32),
            pltpu.VMEM((per_w,), jnp.float32),
        ],
        compiler_params=pltpu.CompilerParams(needs_layout_passes=False),
    )
    def k(rank_hbm, w_hbm, gate_hbm, w_v, r_v, g_v):
        wid = lax.axis_index("s") * NC + lax.axis_index("c")
        base = wid * per_w
        pltpu.sync_copy(w_hbm, w_v)
        pltpu.sync_copy(rank_hbm.at[pl.ds(base, per_w)], r_v)
        for v in range(per_w // L):
            idx = r_v[pl.ds(v * L, L)]
            g_v[pl.ds(v * L, L)] = plsc.load_gather(w_v, [idx])
        pltpu.sync_copy(g_v, gate_hbm.at[pl.ds(base, per_w)])

    return k(rank_flat, w)


def kernel(x, atten, W):
    B, S, D = x.shape
    BH = atten.shape[0]
    SB = 512
    nsb = S // SB
    nsteps = BH + B * nsb

    rank = pl.pallas_call(
        _make_reduce_rank_body(BH, SB, nsb),
        grid=(nsteps,),
        in_specs=[
            pl.BlockSpec(
                (1, S, S), lambda t, _m=BH - 1: (jnp.minimum(t, _m), 0, 0)),
        ],
        out_specs=pl.BlockSpec(
            (SB, 1), lambda t, _b=BH: (jnp.maximum(t - _b, 0), 0)),
        out_shape=jax.ShapeDtypeStruct((B * S, 1), jnp.int32),
        scratch_shapes=[pltpu.VMEM((B, S), jnp.float32)],
    )(atten)

    gate = _sc_gather(rank.reshape(B * S), W)
    gate_col = gate.reshape(B * S, 1)

    out = pl.pallas_call(
        _mul_body,
        grid=(B, nsb),
        in_specs=[
            pl.BlockSpec((SB, 1), lambda b, s, _n=nsb: (b * _n + s, 0)),
            pl.BlockSpec((1, SB, D), lambda b, s: (b, s, 0)),
        ],
        out_specs=pl.BlockSpec((1, SB, D), lambda b, s: (b, s, 0)),
        out_shape=jax.ShapeDtypeStruct((B, S, D), jnp.float32),
    )(gate_col, x)
    return out


# speedup vs baseline: 1.0607x; 1.0445x over previous
import functools

import jax
import jax.numpy as jnp
from jax import lax
from jax.experimental import pallas as pl
from jax.experimental.pallas import tpu as pltpu
from jax.experimental.pallas import tpu_sc as plsc

_HEADS = 12


def _make_reduce_rank_body(BH, SB, nsb):
    def body(a_ref, rank_ref, acc_ref):
        t = pl.program_id(0)

        @pl.when(t < BH)
        def _phase_a():
            m = t
            b = m // _HEADS
            data = a_ref[0]
            R, S = data.shape
            rows = jax.lax.broadcasted_iota(jnp.int32, (R, S), 0)
            cols = jax.lax.broadcasted_iota(jnp.int32, (R, S), 1)
            contrib = jnp.where(rows == cols, 0.0, data)
            part = jnp.sum(contrib, axis=0, keepdims=True)

            @pl.when(m % _HEADS == 0)
            def _():
                acc_ref[pl.ds(b, 1), :] = part

            @pl.when(m % _HEADS != 0)
            def _():
                acc_ref[pl.ds(b, 1), :] += part

        @pl.when(t >= BH)
        def _phase_b():
            i = t - BH
            b = i // nsb
            sblk = i % nsb
            S = acc_ref.shape[1]
            a_row = acc_ref[pl.ds(b, 1), :]
            a_col = jnp.transpose(
                acc_ref[pl.ds(b, 1), pl.ds(sblk * SB, SB)])
            s_glob = jax.lax.broadcasted_iota(jnp.int32, (SB, S), 0) + sblk * SB
            j_glob = jax.lax.broadcasted_iota(jnp.int32, (SB, S), 1)
            gt = a_row > a_col
            tie = jnp.logical_and(a_row == a_col, j_glob < s_glob)
            cmp = jnp.where(jnp.logical_or(gt, tie), 1.0, 0.0)
            rcol = jnp.sum(cmp, axis=1, keepdims=True).astype(jnp.int32)
            rank_ref[...] = jnp.transpose(rcol)

    return body


def _make_mul_body(SB, nsb):
    def body(gate_ref, x_ref, out_ref):
        i = pl.program_id(0) * nsb + pl.program_id(1)
        gate = jnp.transpose(gate_ref[:, pl.ds(i * SB, SB)])
        out_ref[0] = x_ref[0] * gate

    return body


def _sc_gather(rank_flat, w):
    BS = rank_flat.shape[0]
    S = w.shape[0]
    info = plsc.get_sparse_core_info()
    NC, NS, L = info.num_cores, info.num_subcores, info.num_lanes
    NW = NC * NS
    per_w = BS // NW

    mesh = plsc.VectorSubcoreMesh(core_axis_name="c", subcore_axis_name="s")

    @functools.partial(
        pl.kernel,
        out_type=jax.ShapeDtypeStruct((BS,), jnp.float32),
        mesh=mesh,
        scratch_types=[
            pltpu.VMEM((S,), jnp.float32),
            pltpu.VMEM((per_w,), jnp.int32),
            pltpu.VMEM((per_w,), jnp.float32),
        ],
        compiler_params=pltpu.CompilerParams(needs_layout_passes=False),
    )
    def k(rank_hbm, w_hbm, gate_hbm, w_v, r_v, g_v):
        wid = lax.axis_index("s") * NC + lax.axis_index("c")
        base = wid * per_w
        pltpu.sync_copy(w_hbm, w_v)
        pltpu.sync_copy(rank_hbm.at[pl.ds(base, per_w)], r_v)
        for v in range(per_w // L):
            idx = r_v[pl.ds(v * L, L)]
            g_v[pl.ds(v * L, L)] = plsc.load_gather(w_v, [idx])
        pltpu.sync_copy(g_v, gate_hbm.at[pl.ds(base, per_w)])

    return k(rank_flat, w)


def kernel(x, atten, W):
    B, S, D = x.shape
    BH = atten.shape[0]
    SB = 512
    nsb = S // SB
    nsteps = BH + B * nsb

    rank = pl.pallas_call(
        _make_reduce_rank_body(BH, SB, nsb),
        grid=(nsteps,),
        in_specs=[
            pl.BlockSpec(
                (1, S, S), lambda t, _m=BH - 1: (jnp.minimum(t, _m), 0, 0)),
        ],
        out_specs=pl.BlockSpec(
            (1, SB), lambda t, _b=BH: (0, jnp.maximum(t - _b, 0))),
        out_shape=jax.ShapeDtypeStruct((1, B * S), jnp.int32),
        scratch_shapes=[pltpu.VMEM((B, S), jnp.float32)],
    )(atten)

    gate = _sc_gather(rank.reshape(B * S), W)
    gate_row = gate.reshape(1, B * S)

    out = pl.pallas_call(
        _make_mul_body(SB, nsb),
        grid=(B, nsb),
        in_specs=[
            pl.BlockSpec((1, B * S), lambda b, s: (0, 0)),
            pl.BlockSpec((1, SB, D), lambda b, s: (b, s, 0)),
        ],
        out_specs=pl.BlockSpec((1, SB, D), lambda b, s: (b, s, 0)),
        out_shape=jax.ShapeDtypeStruct((B, S, D), jnp.float32),
    )(gate_row, x)
    return out
